# baseline (speedup 1.0000x reference)
import functools

import jax
import jax.numpy as jnp
from jax import lax
from jax.experimental import pallas as pl
from jax.experimental.pallas import tpu as pltpu
from jax.experimental.pallas import tpu_sc as plsc

IGNORE_INDEX = 255
THRESH = 0.7
MIN_KEPT = 100000

THRESH_BITS = 0x3F333333
ONE_BITS = 0x3F800000
L = 16
UNROLL = 4
NWIN = 8


def _softmax_kernel(*refs, c):
    preds = refs[:NWIN]
    tgts = refs[NWIN:2 * NWIN]
    bits_ref, nll_ref = refs[2 * NWIN:]

    for w in range(NWIN):
        x = preds[w][0]
        lab = tgts[w][0]

        m = jnp.max(x, axis=0, keepdims=True)
        e = jnp.exp(x - m)
        s = jnp.sum(e, axis=0, keepdims=True)
        cls = jax.lax.broadcasted_iota(jnp.int32, (c, x.shape[1]), 0)
        sel = cls == lab
        xl = jnp.sum(jnp.where(sel, x, 0.0), axis=0, keepdims=True)
        prd = jnp.exp(xl - m) / s
        nll = jnp.log(s) - (xl - m)

        b = x.shape[1]
        bits_ref[pl.ds(w, 1)] = (
            jax.lax.bitcast_convert_type(prd, jnp.int32).reshape(1, 1, b))
        nll_ref[pl.ds(w, 1)] = nll.reshape(1, 1, b)


def _make_sc_scan(total):
    info = plsc.get_sparse_core_info()
    nc, ns = info.num_cores, info.num_subcores
    chunk = total // (nc * ns)
    mesh = plsc.VectorSubcoreMesh(core_axis_name="c", subcore_axis_name="s")

    @functools.partial(
        pl.kernel,
        mesh=mesh,
        out_type=[
            jax.ShapeDtypeStruct((nc, ns, L), jnp.float32),
            jax.ShapeDtypeStruct((nc, ns, L), jnp.int32),
        ],
        scratch_types=[
            pltpu.VMEM((chunk,), jnp.int32),
            pltpu.VMEM((chunk,), jnp.float32),
            pltpu.VMEM((L,), jnp.float32),
            pltpu.VMEM((L,), jnp.int32),
        ],
    )
    def sc_scan(bits_hbm, nll_hbm, sum_out, cnt_out,
                bits_v, nll_v, fvec_v, ivec_v):
        cid = lax.axis_index("c")
        sid = lax.axis_index("s")
        ones_i = jnp.ones((L,), jnp.int32)
        zero_i = jnp.zeros((L,), jnp.int32)
        zero_f = jnp.zeros((L,), jnp.float32)
        thr_v = jnp.full((L,), THRESH_BITS, jnp.int32)

        base = (cid * ns + sid) * chunk
        pltpu.sync_copy(bits_hbm.at[pl.ds(base, chunk)], bits_v)
        pltpu.sync_copy(nll_hbm.at[pl.ds(base, chunk)], nll_v)

        def body(i, carry):
            acc_c, acc_s = carry
            for j in range(UNROLL):
                off = (i * UNROLL + j) * L
                v = bits_v[pl.ds(off, L)]
                nl = nll_v[pl.ds(off, L)]
                keep = v <= thr_v
                acc_c = acc_c + jnp.where(keep, ones_i, zero_i)
                acc_s = acc_s + jnp.where(keep, nl, zero_f)
            return acc_c, acc_s

        acc_c, acc_s = lax.fori_loop(0, chunk // (L * UNROLL), body,
                                     (zero_i, zero_f))

        fvec_v[...] = acc_s
        ivec_v[...] = acc_c
        pltpu.sync_copy(fvec_v, sum_out.at[cid, sid])
        pltpu.sync_copy(ivec_v, cnt_out.at[cid, sid])

    return sc_scan


def _combine_kernel(sum_ref, cnt_ref, out_ref):
    s07 = jnp.sum(sum_ref[...])
    c07 = jnp.sum(cnt_ref[...]).astype(jnp.float32)
    fast_loss = s07 / jnp.maximum(c07, 1.0)
    pos = jax.lax.broadcasted_iota(jnp.int32, (1, 2), 1)
    out_ref[...] = jnp.where(pos == 0, fast_loss, c07)


def _slow_kernel(bits_ref, nll_ref, out_ref):
    bits = bits_ref[...]
    nllv = nll_ref[...]

    def body(_, carry):
        lo, hi = carry
        mid = jax.lax.div(lo + hi, 2)
        cnt = jnp.sum((bits <= mid).astype(jnp.int32))
        ge = cnt >= MIN_KEPT
        return (jnp.where(ge, lo, mid + 1), jnp.where(ge, mid, hi))

    _, thr = jax.lax.fori_loop(
        0, 23, body, (jnp.int32(THRESH_BITS + 1), jnp.int32(ONE_BITS)))
    kept = bits <= thr
    cntk = jnp.sum(kept.astype(jnp.float32))
    snll = jnp.sum(jnp.where(kept, nllv, 0.0))
    loss = snll / jnp.maximum(cntk, 1.0)
    out_ref[...] = jnp.full((1, 1), loss, dtype=jnp.float32)


@jax.jit
def kernel(predict, target):
    n, c, h, w = predict.shape
    hw = h * w
    blk = min(2048, hw)
    assert hw % blk == 0
    blocks_per_n = hw // blk
    nblocks = n * blocks_per_n
    assert nblocks % NWIN == 0
    grid = nblocks // NWIN
    total = n * hw

    predict3 = predict.reshape(n, c, hw)
    target3 = target.reshape(n, 1, hw).astype(jnp.int32)

    def pspec(w):
        return pl.BlockSpec(
            (1, c, blk),
            lambda g, w=w: ((g * NWIN + w) // blocks_per_n, 0,
                            (g * NWIN + w) % blocks_per_n))

    def tspec(w):
        return pl.BlockSpec(
            (1, 1, blk),
            lambda g, w=w: ((g * NWIN + w) // blocks_per_n, 0,
                            (g * NWIN + w) % blocks_per_n))

    bits, nll = pl.pallas_call(
        functools.partial(_softmax_kernel, c=c),
        grid=(grid,),
        in_specs=[pspec(w) for w in range(NWIN)]
        + [tspec(w) for w in range(NWIN)],
        out_specs=[
            pl.BlockSpec((NWIN, 1, blk), lambda g: (g, 0, 0)),
            pl.BlockSpec((NWIN, 1, blk), lambda g: (g, 0, 0)),
        ],
        out_shape=[
            jax.ShapeDtypeStruct((nblocks, 1, blk), jnp.int32),
            jax.ShapeDtypeStruct((nblocks, 1, blk), jnp.float32),
        ],
    )(*([predict3] * NWIN + [target3] * NWIN))

    sums, cnts = _make_sc_scan(total)(bits.reshape(total), nll.reshape(total))

    comb = pl.pallas_call(
        _combine_kernel,
        out_shape=jax.ShapeDtypeStruct((1, 2), jnp.float32),
    )(sums, cnts)
    fast_loss = comb[0, 0]
    c07 = comb[0, 1]

    def fast(_):
        return fast_loss

    def slow(_):
        out = pl.pallas_call(
            _slow_kernel,
            out_shape=jax.ShapeDtypeStruct((1, 1), jnp.float32),
        )(bits, nll)
        return out[0, 0]

    loss = lax.cond(c07 >= MIN_KEPT, fast, slow, 0)
    return loss.reshape(())


# speedup vs baseline: 1.0239x; 1.0239x over previous
import functools

import jax
import jax.numpy as jnp
from jax import lax
from jax.experimental import pallas as pl
from jax.experimental.pallas import tpu as pltpu
from jax.experimental.pallas import tpu_sc as plsc

IGNORE_INDEX = 255
THRESH = 0.7
MIN_KEPT = 100000

THRESH_BITS = 0x3F333333
ONE_BITS = 0x3F800000
L = 16
UNROLL = 4
NWIN = 16


def _softmax_kernel(*refs, c):
    preds = refs[:NWIN]
    tgts = refs[NWIN:2 * NWIN]
    bits_ref, nll_ref = refs[2 * NWIN:]

    for w in range(NWIN):
        x = preds[w][0]
        lab = tgts[w][0]

        m = jnp.max(x, axis=0, keepdims=True)
        e = jnp.exp(x - m)
        s = jnp.sum(e, axis=0, keepdims=True)
        cls = jax.lax.broadcasted_iota(jnp.int32, (c, x.shape[1]), 0)
        sel = cls == lab
        xl = jnp.sum(jnp.where(sel, x, 0.0), axis=0, keepdims=True)
        prd = jnp.exp(xl - m) / s
        nll = jnp.log(s) - (xl - m)

        b = x.shape[1]
        bits_ref[pl.ds(w, 1)] = (
            jax.lax.bitcast_convert_type(prd, jnp.int32).reshape(1, 1, b))
        nll_ref[pl.ds(w, 1)] = nll.reshape(1, 1, b)


def _make_sc_scan(total):
    info = plsc.get_sparse_core_info()
    nc, ns = info.num_cores, info.num_subcores
    chunk = total // (nc * ns)
    mesh = plsc.VectorSubcoreMesh(core_axis_name="c", subcore_axis_name="s")

    @functools.partial(
        pl.kernel,
        mesh=mesh,
        out_type=[
            jax.ShapeDtypeStruct((nc, ns, L), jnp.float32),
            jax.ShapeDtypeStruct((nc, ns, L), jnp.int32),
        ],
        scratch_types=[
            pltpu.VMEM((chunk,), jnp.int32),
            pltpu.VMEM((chunk,), jnp.float32),
            pltpu.VMEM((L,), jnp.float32),
            pltpu.VMEM((L,), jnp.int32),
        ],
    )
    def sc_scan(bits_hbm, nll_hbm, sum_out, cnt_out,
                bits_v, nll_v, fvec_v, ivec_v):
        cid = lax.axis_index("c")
        sid = lax.axis_index("s")
        ones_i = jnp.ones((L,), jnp.int32)
        zero_i = jnp.zeros((L,), jnp.int32)
        zero_f = jnp.zeros((L,), jnp.float32)
        thr_v = jnp.full((L,), THRESH_BITS, jnp.int32)

        base = (cid * ns + sid) * chunk
        pltpu.sync_copy(bits_hbm.at[pl.ds(base, chunk)], bits_v)
        pltpu.sync_copy(nll_hbm.at[pl.ds(base, chunk)], nll_v)

        def body(i, carry):
            acc_c, acc_s = carry
            for j in range(UNROLL):
                off = (i * UNROLL + j) * L
                v = bits_v[pl.ds(off, L)]
                nl = nll_v[pl.ds(off, L)]
                keep = v <= thr_v
                acc_c = acc_c + jnp.where(keep, ones_i, zero_i)
                acc_s = acc_s + jnp.where(keep, nl, zero_f)
            return acc_c, acc_s

        acc_c, acc_s = lax.fori_loop(0, chunk // (L * UNROLL), body,
                                     (zero_i, zero_f))

        fvec_v[...] = acc_s
        ivec_v[...] = acc_c
        pltpu.sync_copy(fvec_v, sum_out.at[cid, sid])
        pltpu.sync_copy(ivec_v, cnt_out.at[cid, sid])

    return sc_scan


def _combine_kernel(sum_ref, cnt_ref, out_ref):
    s07 = jnp.sum(sum_ref[...])
    c07 = jnp.sum(cnt_ref[...]).astype(jnp.float32)
    fast_loss = s07 / jnp.maximum(c07, 1.0)
    pos = jax.lax.broadcasted_iota(jnp.int32, (1, 2), 1)
    out_ref[...] = jnp.where(pos == 0, fast_loss, c07)


def _slow_kernel(bits_ref, nll_ref, out_ref):
    bits = bits_ref[...]
    nllv = nll_ref[...]

    def body(_, carry):
        lo, hi = carry
        mid = jax.lax.div(lo + hi, 2)
        cnt = jnp.sum((bits <= mid).astype(jnp.int32))
        ge = cnt >= MIN_KEPT
        return (jnp.where(ge, lo, mid + 1), jnp.where(ge, mid, hi))

    _, thr = jax.lax.fori_loop(
        0, 23, body, (jnp.int32(THRESH_BITS + 1), jnp.int32(ONE_BITS)))
    kept = bits <= thr
    cntk = jnp.sum(kept.astype(jnp.float32))
    snll = jnp.sum(jnp.where(kept, nllv, 0.0))
    loss = snll / jnp.maximum(cntk, 1.0)
    out_ref[...] = jnp.full((1, 1), loss, dtype=jnp.float32)


@jax.jit
def kernel(predict, target):
    n, c, h, w = predict.shape
    hw = h * w
    blk = min(2048, hw)
    assert hw % blk == 0
    blocks_per_n = hw // blk
    nblocks = n * blocks_per_n
    assert nblocks % NWIN == 0
    grid = nblocks // NWIN
    total = n * hw

    predict3 = predict.reshape(n, c, hw)
    target3 = target.reshape(n, 1, hw).astype(jnp.int32)

    def pspec(w):
        return pl.BlockSpec(
            (1, c, blk),
            lambda g, w=w: ((g * NWIN + w) // blocks_per_n, 0,
                            (g * NWIN + w) % blocks_per_n))

    def tspec(w):
        return pl.BlockSpec(
            (1, 1, blk),
            lambda g, w=w: ((g * NWIN + w) // blocks_per_n, 0,
                            (g * NWIN + w) % blocks_per_n))

    bits, nll = pl.pallas_call(
        functools.partial(_softmax_kernel, c=c),
        grid=(grid,),
        in_specs=[pspec(w) for w in range(NWIN)]
        + [tspec(w) for w in range(NWIN)],
        out_specs=[
            pl.BlockSpec((NWIN, 1, blk), lambda g: (g, 0, 0)),
            pl.BlockSpec((NWIN, 1, blk), lambda g: (g, 0, 0)),
        ],
        out_shape=[
            jax.ShapeDtypeStruct((nblocks, 1, blk), jnp.int32),
            jax.ShapeDtypeStruct((nblocks, 1, blk), jnp.float32),
        ],
    )(*([predict3] * NWIN + [target3] * NWIN))

    sums, cnts = _make_sc_scan(total)(bits.reshape(total), nll.reshape(total))

    comb = pl.pallas_call(
        _combine_kernel,
        out_shape=jax.ShapeDtypeStruct((1, 2), jnp.float32),
    )(sums, cnts)
    fast_loss = comb[0, 0]
    c07 = comb[0, 1]

    def fast(_):
        return fast_loss

    def slow(_):
        out = pl.pallas_call(
            _slow_kernel,
            out_shape=jax.ShapeDtypeStruct((1, 1), jnp.float32),
        )(bits, nll)
        return out[0, 0]

    loss = lax.cond(c07 >= MIN_KEPT, fast, slow, 0)
    return loss.reshape(())
